# baseline (speedup 1.0000x reference)
import jax
import jax.numpy as jnp
from jax import lax
from jax.experimental import pallas as pl
from jax.experimental.pallas import tpu as pltpu


def _rvq_body(x_ref, cb_ref, out_ref, q1_ref, q2_ref, loss_ref,
              eth_ref, etm_ref, etl_ref, e2_ref, lacc_ref, cbb_ref):
    nq, n, k = cb_ref.shape
    bt = x_ref.shape[2]

    @pl.when((pl.program_id(0) == 0) & (pl.program_id(1) == 0))
    def _init():
        lacc_ref[...] = jnp.zeros_like(lacc_ref)
        for i in range(nq):
            cb = cb_ref[i]
            cbt = cb.T
            hi = cbt.astype(jnp.bfloat16)
            rem = cbt - hi.astype(jnp.float32)
            mid = rem.astype(jnp.bfloat16)
            lo = (rem - mid.astype(jnp.float32)).astype(jnp.bfloat16)
            eth_ref[i] = hi
            etm_ref[i] = mid
            etl_ref[i] = lo
            e2 = jnp.sum(cb * cb, axis=1, keepdims=True)
            e2_ref[i] = jnp.broadcast_to(e2, (n, bt))
            cbb_ref[i] = cb.astype(jnp.bfloat16)

    r = x_ref[0]
    acc = jnp.zeros_like(r)
    iota_n = lax.broadcasted_iota(jnp.int32, (n, bt), 0)
    ch = 64
    iota_c = lax.broadcasted_iota(jnp.int32, (ch, bt), 0)
    for i in range(nq):
        s = jnp.sum(r * r, axis=0, keepdims=True)
        m = jnp.dot(cbb_ref[i], r.astype(jnp.bfloat16),
                    preferred_element_type=jnp.float32)
        best = None
        bidx = None
        for c in range(0, n, ch):
            xc = (s - 2.0 * m[c:c + ch, :]) + e2_ref[i, c:c + ch, :]
            cmn = jnp.min(xc, axis=0, keepdims=True)
            cidx = jnp.min(jnp.where(xc == cmn, iota_c + c, n),
                           axis=0, keepdims=True)
            if best is None:
                best, bidx = cmn, cidx
            else:
                take = cmn < best
                best = jnp.where(take, cmn, best)
                bidx = jnp.where(take, cidx, bidx)
        oh = (iota_n == bidx).astype(jnp.bfloat16)
        q = ((jnp.dot(eth_ref[i], oh, preferred_element_type=jnp.float32)
              + jnp.dot(etm_ref[i], oh, preferred_element_type=jnp.float32))
             + jnp.dot(etl_ref[i], oh, preferred_element_type=jnp.float32))
        d = q - r
        lacc_ref[i:i + 1, :] += jnp.sum(d * d, axis=0, keepdims=True)
        q_ste = r + d
        acc = acc + q_ste
        r = r - q_ste
        if i == 0:
            q1_ref[0] = q_ste
        elif i == 1:
            q2_ref[0] = q_ste
    out_ref[0] = acc

    @pl.when((pl.program_id(0) == pl.num_programs(0) - 1)
             & (pl.program_id(1) == pl.num_programs(1) - 1))
    def _final():
        loss_ref[...] = jnp.sum(lacc_ref[...], axis=1, keepdims=True)


def kernel(x, codebooks):
    b, d, t = x.shape
    nq, n, k = codebooks.shape
    bt = t if t < 512 else 512
    grid = (b, t // bt)

    out, q1, q2, loss_sums = pl.pallas_call(
        _rvq_body,
        grid=grid,
        in_specs=[
            pl.BlockSpec((1, d, bt), lambda bi, ti: (bi, 0, ti)),
            pl.BlockSpec((nq, n, k), lambda bi, ti: (0, 0, 0)),
        ],
        out_specs=[
            pl.BlockSpec((1, d, bt), lambda bi, ti: (bi, 0, ti)),
            pl.BlockSpec((1, d, bt), lambda bi, ti: (bi, 0, ti)),
            pl.BlockSpec((1, d, bt), lambda bi, ti: (bi, 0, ti)),
            pl.BlockSpec((nq, 1), lambda bi, ti: (0, 0)),
        ],
        out_shape=[
            jax.ShapeDtypeStruct((b, d, t), x.dtype),
            jax.ShapeDtypeStruct((b, d, t), x.dtype),
            jax.ShapeDtypeStruct((b, d, t), x.dtype),
            jax.ShapeDtypeStruct((nq, 1), jnp.float32),
        ],
        scratch_shapes=[
            pltpu.VMEM((nq, k, n), jnp.bfloat16),
            pltpu.VMEM((nq, k, n), jnp.bfloat16),
            pltpu.VMEM((nq, k, n), jnp.bfloat16),
            pltpu.VMEM((nq, n, bt), jnp.float32),
            pltpu.VMEM((nq, bt), jnp.float32),
            pltpu.VMEM((nq, n, k), jnp.bfloat16),
        ],
    )(x, codebooks)

    count = b * d * t
    com = jnp.mean(loss_sums[:, 0] / count)
    return out, q1, q2, com


# speedup vs baseline: 1.0065x; 1.0065x over previous
import jax
import jax.numpy as jnp
from jax import lax
from jax.experimental import pallas as pl
from jax.experimental.pallas import tpu as pltpu


def _rvq_body(x_ref, cb_ref, out_ref, q1_ref, q2_ref, loss_ref,
              eth_ref, etm_ref, etl_ref, e2_ref, lacc_ref, cbb_ref):
    nq, n, k = cb_ref.shape
    bt = x_ref.shape[2]

    @pl.when((pl.program_id(0) == 0) & (pl.program_id(1) == 0))
    def _init():
        lacc_ref[...] = jnp.zeros_like(lacc_ref)
        for i in range(nq):
            cb = cb_ref[i]
            cbt = cb.T
            hi = cbt.astype(jnp.bfloat16)
            rem = cbt - hi.astype(jnp.float32)
            mid = rem.astype(jnp.bfloat16)
            lo = (rem - mid.astype(jnp.float32)).astype(jnp.bfloat16)
            eth_ref[i] = hi
            etm_ref[i] = mid
            etl_ref[i] = lo
            e2 = jnp.sum(cb * cb, axis=1, keepdims=True)
            e2_ref[i] = jnp.broadcast_to(e2, (n, bt))
            cbb_ref[i] = cb.astype(jnp.bfloat16)

    hb = bt // 2
    off = (0, hb)
    x0 = [x_ref[0, :, 0:hb], x_ref[0, :, hb:bt]]
    r = list(x0)
    iota_n = lax.broadcasted_iota(jnp.int32, (n, hb), 0)
    ch = 64
    iota_c = lax.broadcasted_iota(jnp.int32, (ch, hb), 0)
    for i in range(nq):
        m = [jnp.dot(cbb_ref[i], r[hh].astype(jnp.bfloat16),
                     preferred_element_type=jnp.float32) for hh in range(2)]
        s = [jnp.sum(r[hh] * r[hh], axis=0, keepdims=True) for hh in range(2)]
        bidx = []
        for hh in range(2):
            best = None
            bi = None
            for c in range(0, n, ch):
                xc = ((s[hh] - 2.0 * m[hh][c:c + ch, :])
                      + e2_ref[i, c:c + ch, off[hh]:off[hh] + hb])
                cmn = jnp.min(xc, axis=0, keepdims=True)
                cidx = jnp.min(jnp.where(xc == cmn, iota_c + c, n),
                               axis=0, keepdims=True)
                if best is None:
                    best, bi = cmn, cidx
                else:
                    take = cmn < best
                    best = jnp.where(take, cmn, best)
                    bi = jnp.where(take, cidx, bi)
            bidx.append(bi)
        oh = [(iota_n == bidx[hh]).astype(jnp.bfloat16) for hh in range(2)]
        q = [((jnp.dot(eth_ref[i], oh[hh], preferred_element_type=jnp.float32)
               + jnp.dot(etm_ref[i], oh[hh], preferred_element_type=jnp.float32))
              + jnp.dot(etl_ref[i], oh[hh], preferred_element_type=jnp.float32))
             for hh in range(2)]
        for hh in range(2):
            d = q[hh] - r[hh]
            lacc_ref[i:i + 1, off[hh]:off[hh] + hb] += jnp.sum(
                d * d, axis=0, keepdims=True)
            q_ste = r[hh] + d
            r[hh] = r[hh] - q_ste
            if i == 0:
                q1_ref[0, :, off[hh]:off[hh] + hb] = q_ste
            elif i == 1:
                q2_ref[0, :, off[hh]:off[hh] + hb] = q_ste
    for hh in range(2):
        out_ref[0, :, off[hh]:off[hh] + hb] = x0[hh] - r[hh]

    @pl.when((pl.program_id(0) == pl.num_programs(0) - 1)
             & (pl.program_id(1) == pl.num_programs(1) - 1))
    def _final():
        loss_ref[...] = jnp.sum(lacc_ref[...], axis=1, keepdims=True)


def kernel(x, codebooks):
    b, d, t = x.shape
    nq, n, k = codebooks.shape
    bt = t if t < 512 else 512
    grid = (b, t // bt)

    out, q1, q2, loss_sums = pl.pallas_call(
        _rvq_body,
        grid=grid,
        in_specs=[
            pl.BlockSpec((1, d, bt), lambda bi, ti: (bi, 0, ti)),
            pl.BlockSpec((nq, n, k), lambda bi, ti: (0, 0, 0)),
        ],
        out_specs=[
            pl.BlockSpec((1, d, bt), lambda bi, ti: (bi, 0, ti)),
            pl.BlockSpec((1, d, bt), lambda bi, ti: (bi, 0, ti)),
            pl.BlockSpec((1, d, bt), lambda bi, ti: (bi, 0, ti)),
            pl.BlockSpec((nq, 1), lambda bi, ti: (0, 0)),
        ],
        out_shape=[
            jax.ShapeDtypeStruct((b, d, t), x.dtype),
            jax.ShapeDtypeStruct((b, d, t), x.dtype),
            jax.ShapeDtypeStruct((b, d, t), x.dtype),
            jax.ShapeDtypeStruct((nq, 1), jnp.float32),
        ],
        scratch_shapes=[
            pltpu.VMEM((nq, k, n), jnp.bfloat16),
            pltpu.VMEM((nq, k, n), jnp.bfloat16),
            pltpu.VMEM((nq, k, n), jnp.bfloat16),
            pltpu.VMEM((nq, n, bt), jnp.float32),
            pltpu.VMEM((nq, bt), jnp.float32),
            pltpu.VMEM((nq, n, k), jnp.bfloat16),
        ],
    )(x, codebooks)

    count = b * d * t
    com = jnp.mean(loss_sums[:, 0] / count)
    return out, q1, q2, com


# speedup vs baseline: 1.0270x; 1.0204x over previous
import jax
import jax.numpy as jnp
from jax import lax
from jax.experimental import pallas as pl
from jax.experimental.pallas import tpu as pltpu


def _rvq_body(x_ref, cb_ref, out_ref, q1_ref, q2_ref, loss_ref,
              eth_ref, etm_ref, etl_ref, e2_ref, lacc_ref, cbb_ref):
    nq, n, k = cb_ref.shape
    bt = x_ref.shape[2]

    @pl.when((pl.program_id(0) == 0) & (pl.program_id(1) == 0))
    def _init():
        lacc_ref[...] = jnp.zeros_like(lacc_ref)
        for i in range(nq):
            cb = cb_ref[i]
            cbt = cb.T
            hi = cbt.astype(jnp.bfloat16)
            rem = cbt - hi.astype(jnp.float32)
            mid = rem.astype(jnp.bfloat16)
            lo = (rem - mid.astype(jnp.float32)).astype(jnp.bfloat16)
            eth_ref[i] = hi
            etm_ref[i] = mid
            etl_ref[i] = lo
            e2 = jnp.sum(cb * cb, axis=1, keepdims=True)
            e2_ref[i] = jnp.broadcast_to(e2, (n, bt))
            cbb_ref[i] = cb.astype(jnp.bfloat16)

    hb = bt // 2
    off = (0, hb)
    x0 = [x_ref[0, :, 0:hb], x_ref[0, :, hb:bt]]
    r = list(x0)
    iota_n = lax.broadcasted_iota(jnp.int32, (n, hb), 0)
    ch = 128
    iota_c = lax.broadcasted_iota(jnp.int32, (ch, hb), 0)
    for i in range(nq):
        m = [jnp.dot(cbb_ref[i], r[hh].astype(jnp.bfloat16),
                     preferred_element_type=jnp.float32) for hh in range(2)]
        s = [jnp.sum(r[hh] * r[hh], axis=0, keepdims=True) for hh in range(2)]
        bidx = []
        for hh in range(2):
            best = None
            bi = None
            for c in range(0, n, ch):
                xc = ((s[hh] - 2.0 * m[hh][c:c + ch, :])
                      + e2_ref[i, c:c + ch, off[hh]:off[hh] + hb])
                cmn = jnp.min(xc, axis=0, keepdims=True)
                cidx = jnp.min(jnp.where(xc == cmn, iota_c + c, n),
                               axis=0, keepdims=True)
                if best is None:
                    best, bi = cmn, cidx
                else:
                    take = cmn < best
                    best = jnp.where(take, cmn, best)
                    bi = jnp.where(take, cidx, bi)
            bidx.append(bi)
        oh = [(iota_n == bidx[hh]).astype(jnp.bfloat16) for hh in range(2)]
        q = [((jnp.dot(eth_ref[i], oh[hh], preferred_element_type=jnp.float32)
               + jnp.dot(etm_ref[i], oh[hh], preferred_element_type=jnp.float32))
              + jnp.dot(etl_ref[i], oh[hh], preferred_element_type=jnp.float32))
             for hh in range(2)]
        for hh in range(2):
            d = q[hh] - r[hh]
            lacc_ref[i:i + 1, off[hh]:off[hh] + hb] += jnp.sum(
                d * d, axis=0, keepdims=True)
            q_ste = r[hh] + d
            r[hh] = r[hh] - q_ste
            if i == 0:
                q1_ref[0, :, off[hh]:off[hh] + hb] = q_ste
            elif i == 1:
                q2_ref[0, :, off[hh]:off[hh] + hb] = q_ste
    for hh in range(2):
        out_ref[0, :, off[hh]:off[hh] + hb] = x0[hh] - r[hh]

    @pl.when((pl.program_id(0) == pl.num_programs(0) - 1)
             & (pl.program_id(1) == pl.num_programs(1) - 1))
    def _final():
        loss_ref[...] = jnp.sum(lacc_ref[...], axis=1, keepdims=True)


def kernel(x, codebooks):
    b, d, t = x.shape
    nq, n, k = codebooks.shape
    bt = t if t < 512 else 512
    grid = (b, t // bt)

    out, q1, q2, loss_sums = pl.pallas_call(
        _rvq_body,
        grid=grid,
        in_specs=[
            pl.BlockSpec((1, d, bt), lambda bi, ti: (bi, 0, ti)),
            pl.BlockSpec((nq, n, k), lambda bi, ti: (0, 0, 0)),
        ],
        out_specs=[
            pl.BlockSpec((1, d, bt), lambda bi, ti: (bi, 0, ti)),
            pl.BlockSpec((1, d, bt), lambda bi, ti: (bi, 0, ti)),
            pl.BlockSpec((1, d, bt), lambda bi, ti: (bi, 0, ti)),
            pl.BlockSpec((nq, 1), lambda bi, ti: (0, 0)),
        ],
        out_shape=[
            jax.ShapeDtypeStruct((b, d, t), x.dtype),
            jax.ShapeDtypeStruct((b, d, t), x.dtype),
            jax.ShapeDtypeStruct((b, d, t), x.dtype),
            jax.ShapeDtypeStruct((nq, 1), jnp.float32),
        ],
        scratch_shapes=[
            pltpu.VMEM((nq, k, n), jnp.bfloat16),
            pltpu.VMEM((nq, k, n), jnp.bfloat16),
            pltpu.VMEM((nq, k, n), jnp.bfloat16),
            pltpu.VMEM((nq, n, bt), jnp.float32),
            pltpu.VMEM((nq, bt), jnp.float32),
            pltpu.VMEM((nq, n, k), jnp.bfloat16),
        ],
    )(x, codebooks)

    count = b * d * t
    com = jnp.mean(loss_sums[:, 0] / count)
    return out, q1, q2, com


# speedup vs baseline: 1.0475x; 1.0199x over previous
import jax
import jax.numpy as jnp
from jax import lax
from jax.experimental import pallas as pl
from jax.experimental.pallas import tpu as pltpu


def _rvq_body(x_ref, cb_ref, out_ref, q1_ref, q2_ref, loss_ref,
              eth_ref, etm_ref, etl_ref, e2_ref, lacc_ref, cbb_ref):
    nq, n, k = cb_ref.shape
    bt = x_ref.shape[2]

    @pl.when((pl.program_id(0) == 0) & (pl.program_id(1) == 0))
    def _init():
        lacc_ref[...] = jnp.zeros_like(lacc_ref)
        for i in range(nq):
            cb = cb_ref[i]
            cbt = cb.T
            hi = cbt.astype(jnp.bfloat16)
            rem = cbt - hi.astype(jnp.float32)
            mid = rem.astype(jnp.bfloat16)
            lo = (rem - mid.astype(jnp.float32)).astype(jnp.bfloat16)
            eth_ref[i] = hi
            etm_ref[i] = mid
            etl_ref[i] = lo
            e2 = jnp.sum(cb * cb, axis=1, keepdims=True)
            e2_ref[i] = jnp.broadcast_to(e2, (n, bt))
            cbb_ref[i] = cb.astype(jnp.bfloat16)

    hb = bt // 2
    off = (0, hb)
    x0 = [x_ref[0, :, 0:hb], x_ref[0, :, hb:bt]]
    r = list(x0)
    iota_n = lax.broadcasted_iota(jnp.int32, (n, hb), 0)
    ch = 256
    iota_c = lax.broadcasted_iota(jnp.int32, (ch, hb), 0)
    for i in range(nq):
        m = [jnp.dot(cbb_ref[i], r[hh].astype(jnp.bfloat16),
                     preferred_element_type=jnp.float32) for hh in range(2)]
        s = [jnp.sum(r[hh] * r[hh], axis=0, keepdims=True) for hh in range(2)]
        bidx = []
        for hh in range(2):
            best = None
            bi = None
            for c in range(0, n, ch):
                xc = ((s[hh] - 2.0 * m[hh][c:c + ch, :])
                      + e2_ref[i, c:c + ch, off[hh]:off[hh] + hb])
                cmn = jnp.min(xc, axis=0, keepdims=True)
                cidx = jnp.min(jnp.where(xc == cmn, iota_c + c, n),
                               axis=0, keepdims=True)
                if best is None:
                    best, bi = cmn, cidx
                else:
                    take = cmn < best
                    best = jnp.where(take, cmn, best)
                    bi = jnp.where(take, cidx, bi)
            bidx.append(bi)
        oh = [(iota_n == bidx[hh]).astype(jnp.bfloat16) for hh in range(2)]
        q = [((jnp.dot(eth_ref[i], oh[hh], preferred_element_type=jnp.float32)
               + jnp.dot(etm_ref[i], oh[hh], preferred_element_type=jnp.float32))
              + jnp.dot(etl_ref[i], oh[hh], preferred_element_type=jnp.float32))
             for hh in range(2)]
        for hh in range(2):
            d = q[hh] - r[hh]
            lacc_ref[i:i + 1, off[hh]:off[hh] + hb] += jnp.sum(
                d * d, axis=0, keepdims=True)
            q_ste = r[hh] + d
            r[hh] = r[hh] - q_ste
            if i == 0:
                q1_ref[0, :, off[hh]:off[hh] + hb] = q_ste
            elif i == 1:
                q2_ref[0, :, off[hh]:off[hh] + hb] = q_ste
    for hh in range(2):
        out_ref[0, :, off[hh]:off[hh] + hb] = x0[hh] - r[hh]

    @pl.when((pl.program_id(0) == pl.num_programs(0) - 1)
             & (pl.program_id(1) == pl.num_programs(1) - 1))
    def _final():
        loss_ref[...] = jnp.sum(lacc_ref[...], axis=1, keepdims=True)


def kernel(x, codebooks):
    b, d, t = x.shape
    nq, n, k = codebooks.shape
    bt = t if t < 512 else 512
    grid = (b, t // bt)

    out, q1, q2, loss_sums = pl.pallas_call(
        _rvq_body,
        grid=grid,
        in_specs=[
            pl.BlockSpec((1, d, bt), lambda bi, ti: (bi, 0, ti)),
            pl.BlockSpec((nq, n, k), lambda bi, ti: (0, 0, 0)),
        ],
        out_specs=[
            pl.BlockSpec((1, d, bt), lambda bi, ti: (bi, 0, ti)),
            pl.BlockSpec((1, d, bt), lambda bi, ti: (bi, 0, ti)),
            pl.BlockSpec((1, d, bt), lambda bi, ti: (bi, 0, ti)),
            pl.BlockSpec((nq, 1), lambda bi, ti: (0, 0)),
        ],
        out_shape=[
            jax.ShapeDtypeStruct((b, d, t), x.dtype),
            jax.ShapeDtypeStruct((b, d, t), x.dtype),
            jax.ShapeDtypeStruct((b, d, t), x.dtype),
            jax.ShapeDtypeStruct((nq, 1), jnp.float32),
        ],
        scratch_shapes=[
            pltpu.VMEM((nq, k, n), jnp.bfloat16),
            pltpu.VMEM((nq, k, n), jnp.bfloat16),
            pltpu.VMEM((nq, k, n), jnp.bfloat16),
            pltpu.VMEM((nq, n, bt), jnp.float32),
            pltpu.VMEM((nq, bt), jnp.float32),
            pltpu.VMEM((nq, n, k), jnp.bfloat16),
        ],
    )(x, codebooks)

    count = b * d * t
    com = jnp.mean(loss_sums[:, 0] / count)
    return out, q1, q2, com
